# baseline (speedup 1.0000x reference)
import functools

import jax
import jax.numpy as jnp
from jax import lax
from jax.experimental import pallas as pl
from jax.experimental.pallas import tpu as pltpu
from jax.experimental.pallas import tpu_sc as plsc

D_EMBED = 64
NTOKENS = 1000000
BATCH = 4096
HIST = 50
TOTAL = BATCH * HIST
SCALE = 8.0
_TWO23 = 8388608.0

NROW_PAD = 1000064
D_PER_SC = 32
NTILES = 16
B_PER_TILE = BATCH // (2 * NTILES)
VCHUNK = 62464
VTAIL0 = 16 * VCHUNK
VTAIL = NTOKENS - VTAIL0


def _body(x_hbm, lutT_hbm, out_hbm, xb, idxb, gf, g2d, spmem, sem_st, sem_g, sem_w):
    c = lax.axis_index("c")
    s = lax.axis_index("s")

    def xstage(h, _):
        for j in range(2):
            src = x_hbm.at[pl.ds(h * BATCH + s * 256 + j * 128, 128)]
            pltpu.sync_copy(src, xb.at[j, h])
        return 0

    lax.fori_loop(0, HIST, xstage, 0, unroll=False)

    def quant(h, _):
        for j in range(2):
            for l in range(8):
                v = xb[j, h, pl.ds(l * 16, 16)]
                t = v * jnp.float32(NTOKENS - 1)
                r = (t + _TWO23) - _TWO23
                idxb[j, pl.ds(h * 128 + l * 16, 16)] = r.astype(jnp.int32)
        return 0

    lax.fori_loop(0, HIST, quant, 0, unroll=False)

    v0 = s * VCHUNK

    def stage_row(k):
        dd = D_PER_SC * c + k
        cp = pltpu.async_copy(
            lutT_hbm.at[dd, pl.ds(v0, VCHUNK)],
            spmem.at[0, pl.ds(v0, VCHUNK)],
            sem_st,
        )
        @pl.when(s == 15)
        def _tail():
            pltpu.sync_copy(
                lutT_hbm.at[dd, pl.ds(VTAIL0, VTAIL)],
                spmem.at[0, pl.ds(VTAIL0, VTAIL)],
            )

        return cp

    prev_writes = [None, None]
    st = stage_row(0)
    for k in range(D_PER_SC):
        par = k % 2
        st.wait()
        plsc.subcore_barrier()
        gth = [
            pltpu.async_copy(spmem.at[0].at[idxb.at[j]], gf.at[j], sem_g)
            for j in range(2)
        ]
        for g in gth:
            g.wait()
        plsc.subcore_barrier()
        if k + 1 < D_PER_SC:
            st = stage_row(k + 1)
        if prev_writes[par] is not None:
            for w in prev_writes[par]:
                w.wait()

        def scale(h, _):
            for j in range(2):
                for l in range(8):
                    sl = gf[j, pl.ds(h * 128 + l * 16, 16)]
                    g2d[par, j, h, pl.ds(l * 16, 16)] = sl * SCALE
            return 0

        lax.fori_loop(0, HIST, scale, 0, unroll=False)

        dd = D_PER_SC * c + k
        dH = dd // 8
        dL = dd % 8
        prev_writes[par] = [
            pltpu.async_copy(
                g2d.at[par, j], out_hbm.at[:, dH, 2 * s + j, dL, :], sem_w
            )
            for j in range(2)
        ]
    for ws in prev_writes:
        if ws is not None:
            for w in ws:
                w.wait()


_mesh = plsc.VectorSubcoreMesh(core_axis_name="c", subcore_axis_name="s")

_emb = functools.partial(
    pl.kernel,
    out_type=jax.ShapeDtypeStruct((HIST, 8, 32, 8, 128), jnp.float32),
    mesh=_mesh,
    scratch_types=[
        pltpu.VMEM((2, HIST, 128), jnp.float32),
        pltpu.VMEM((2, HIST * 128), jnp.int32),
        pltpu.VMEM((2, HIST * 128), jnp.float32),
        pltpu.VMEM((2, 2, HIST, 128), jnp.float32),
        pltpu.VMEM_SHARED((1, NROW_PAD), jnp.float32),
        pltpu.SemaphoreType.DMA,
        pltpu.SemaphoreType.DMA,
        pltpu.SemaphoreType.DMA,
    ],
    compiler_params=pltpu.CompilerParams(use_tc_tiling_on_sc=False),
)(_body)


def kernel(x, lut):
    x1 = jnp.transpose(x, (1, 2, 0)).reshape(TOTAL)
    lutT = jnp.transpose(lut, (1, 0))
    o = _emb(x1, lutT)
    return jnp.transpose(o, (2, 4, 0, 1, 3)).reshape(BATCH, HIST, D_EMBED)


# speedup vs baseline: 6.5752x; 6.5752x over previous
import functools

import jax
import jax.numpy as jnp
from jax import lax
from jax.experimental import pallas as pl
from jax.experimental.pallas import tpu as pltpu
from jax.experimental.pallas import tpu_sc as plsc

D_EMBED = 64
NTOKENS = 1000000
BATCH = 4096
HIST = 50
TOTAL = BATCH * HIST
SCALE = 8.0

NUM_WORKERS = 32
PER_WORKER = TOTAL // NUM_WORKERS

SUB = 128
NSUB = 5
CHUNK = SUB * NSUB
NCHUNKS = PER_WORKER // CHUNK

_TWO23 = 8388608.0


def _body(x_hbm, lut_hbm, out_hbm, xv, idxv, rows, sem):
    wid = lax.axis_index("s") * 2 + lax.axis_index("c")
    base = wid * PER_WORKER

    def chunk_body(i, carry):
        cbase = base + i * CHUNK
        pltpu.sync_copy(x_hbm.at[pl.ds(cbase, CHUNK)], xv)
        for j in range(CHUNK // 16):
            v = xv[pl.ds(j * 16, 16)]
            t = v * jnp.float32(NTOKENS - 1)
            r = (t + _TWO23) - _TWO23
            s_idx = j // (SUB // 16)
            lane = (j % (SUB // 16)) * 16
            idxv[s_idx, pl.ds(lane, 16)] = r.astype(jnp.int32)
        copies = []
        for s in range(NSUB):
            copies.append(
                pltpu.async_copy(lut_hbm.at[idxv.at[s]], rows.at[s], sem)
            )
        for c in copies:
            c.wait()
        def scale_body(r4, carry2):
            for u in range(4):
                row = r4 * 4 + u
                s_idx = row // SUB
                r_idx = row % SUB
                for k in range(D_EMBED // 16):
                    sl = rows[s_idx, r_idx, pl.ds(k * 16, 16)]
                    rows[s_idx, r_idx, pl.ds(k * 16, 16)] = sl * SCALE
            return carry2

        lax.fori_loop(0, CHUNK // 4, scale_body, 0, unroll=False)
        for s in range(NSUB):
            pltpu.sync_copy(rows.at[s], out_hbm.at[pl.ds(cbase + s * SUB, SUB)])
        return carry

    lax.fori_loop(0, NCHUNKS, chunk_body, 0, unroll=False)


_mesh = plsc.VectorSubcoreMesh(core_axis_name="c", subcore_axis_name="s")

_emb = functools.partial(
    pl.kernel,
    out_type=jax.ShapeDtypeStruct((TOTAL, D_EMBED), jnp.float32),
    mesh=_mesh,
    scratch_types=[
        pltpu.VMEM((CHUNK,), jnp.float32),
        pltpu.VMEM((NSUB, SUB), jnp.int32),
        pltpu.VMEM((NSUB, SUB, D_EMBED), jnp.float32),
        pltpu.SemaphoreType.DMA,
    ],
    compiler_params=pltpu.CompilerParams(use_tc_tiling_on_sc=False),
)(_body)


def kernel(x, lut):
    xf = x.reshape(TOTAL)
    out = _emb(xf, lut)
    return out.reshape(BATCH, HIST, D_EMBED)


# speedup vs baseline: 6.8597x; 1.0433x over previous
import functools

import jax
import jax.numpy as jnp
from jax import lax
from jax.experimental import pallas as pl
from jax.experimental.pallas import tpu as pltpu
from jax.experimental.pallas import tpu_sc as plsc

D_EMBED = 64
NTOKENS = 1000000
BATCH = 4096
HIST = 50
TOTAL = BATCH * HIST
SCALE = 8.0

NUM_WORKERS = 32
PER_WORKER = TOTAL // NUM_WORKERS

SUB = 128
NSUB = 5
CHUNK = SUB * NSUB
NCHUNKS = PER_WORKER // CHUNK

_TWO23 = 8388608.0


def _body(x_hbm, lut_hbm, out_hbm, xv, idxv, rows, sem_g, sem_w):
    wid = lax.axis_index("s") * 2 + lax.axis_index("c")
    base = wid * PER_WORKER

    def quant(i, par):
        pltpu.sync_copy(x_hbm.at[pl.ds(base + i * CHUNK, CHUNK)], xv)
        for j in range(CHUNK // 16):
            v = xv[pl.ds(j * 16, 16)]
            t = v * jnp.float32(NTOKENS - 1)
            r = (t + _TWO23) - _TWO23
            s_idx = j // (SUB // 16)
            lane = (j % (SUB // 16)) * 16
            idxv[par, s_idx, pl.ds(lane, 16)] = r.astype(jnp.int32)

    def fire_gathers(par):
        return [
            pltpu.async_copy(lut_hbm.at[idxv.at[par, s]], rows.at[par, s], sem_g)
            for s in range(NSUB)
        ]

    def fire_writes(i, par):
        return [
            pltpu.async_copy(
                rows.at[par, s],
                out_hbm.at[pl.ds(base + i * CHUNK + s * SUB, SUB)],
                sem_w,
            )
            for s in range(NSUB)
        ]

    def scale(par):
        def body(r4, carry):
            for u in range(4):
                row = r4 * 4 + u
                s_idx = row // SUB
                r_idx = row % SUB
                for k in range(D_EMBED // 16):
                    sl = rows[par, s_idx, r_idx, pl.ds(k * 16, 16)]
                    rows[par, s_idx, r_idx, pl.ds(k * 16, 16)] = sl * SCALE
            return carry

        lax.fori_loop(0, CHUNK // 4, body, 0, unroll=False)

    quant(0, 0)
    gth = fire_gathers(0)
    writes = [None, None]
    for i in range(NCHUNKS):
        par = i % 2
        nxt = 1 - par
        if i + 1 < NCHUNKS:
            quant(i + 1, nxt)
            if writes[nxt] is not None:
                for w in writes[nxt]:
                    w.wait()
        for g in gth:
            g.wait()
        if i + 1 < NCHUNKS:
            gth = fire_gathers(nxt)
        scale(par)
        writes[par] = fire_writes(i, par)
    for ws in writes:
        if ws is not None:
            for w in ws:
                w.wait()


_mesh = plsc.VectorSubcoreMesh(core_axis_name="c", subcore_axis_name="s")

_emb = functools.partial(
    pl.kernel,
    out_type=jax.ShapeDtypeStruct((TOTAL, D_EMBED), jnp.float32),
    mesh=_mesh,
    scratch_types=[
        pltpu.VMEM((CHUNK,), jnp.float32),
        pltpu.VMEM((2, NSUB, SUB), jnp.int32),
        pltpu.VMEM((2, NSUB, SUB, D_EMBED), jnp.float32),
        pltpu.SemaphoreType.DMA,
        pltpu.SemaphoreType.DMA,
    ],
    compiler_params=pltpu.CompilerParams(use_tc_tiling_on_sc=False),
)(_body)


def kernel(x, lut):
    xf = x.reshape(TOTAL)
    out = _emb(xf, lut)
    return out.reshape(BATCH, HIST, D_EMBED)
